# Initial kernel scaffold; baseline (speedup 1.0000x reference)
import functools

import jax
import jax.numpy as jnp
from jax import lax
from jax.experimental import pallas as pl
from jax.experimental.pallas import tpu as pltpu
from jax.experimental.pallas import tpu_sc as plsc

CUTOFF = 5.0
NC, NS, L = 2, 16, 16
NW = NC * NS

_COS_C = (
    0.9999999738948335,
    -4.934800732956998,
    4.058692224683156,
    -1.3351515271358487,
    0.2350219310960258,
    -0.02535563092319045,
    0.0015937868699634932,
)
_SW_C = tuple(0.5 * c for c in _COS_C)
_SW_C = (_SW_C[0] + 0.5,) + _SW_C[1:]


def _make_kernel(n_nodes: int, n_edges: int, chunk: int):
    assert n_edges % (NW * chunk) == 0
    e_per_w = n_edges // NW
    n_chunks = e_per_w // chunk
    groups = chunk // L
    mesh = plsc.VectorSubcoreMesh(core_axis_name="c", subcore_axis_name="s")
    inv_cut2 = 1.0 / (CUTOFF * CUTOFF)
    cut2 = CUTOFF * CUTOFF

    @functools.partial(
        pl.kernel,
        mesh=mesh,
        out_type=[
            jax.ShapeDtypeStruct((n_edges, 3), jnp.float32),
            jax.ShapeDtypeStruct((n_edges,), jnp.float32),
            jax.ShapeDtypeStruct((n_edges,), jnp.float32),
            jax.ShapeDtypeStruct((n_edges,), jnp.int32),
        ],
        scratch_types=[
            pltpu.VMEM((chunk,), jnp.int32),
            pltpu.VMEM((chunk,), jnp.int32),
            pltpu.VMEM((chunk, 4), jnp.float32),
            pltpu.VMEM((chunk, 4), jnp.float32),
            pltpu.VMEM((chunk, 3), jnp.float32),
            pltpu.VMEM((chunk,), jnp.float32),
            pltpu.VMEM((chunk,), jnp.float32),
            pltpu.VMEM((chunk,), jnp.int32),
            pltpu.SemaphoreType.DMA,
            pltpu.SemaphoreType.DMA,
        ],
    )
    def k(coords_hbm, src_hbm, dst_hbm,
          vec_hbm, dist_hbm, sw_hbm, mask_hbm,
          sidx, didx, srow, drow, vec_v, dist_v, sw_v, mask_v, sem_s, sem_d):
        wid = lax.axis_index("s") * NC + lax.axis_index("c")
        base0 = wid * e_per_w

        col0 = jnp.zeros((L,), jnp.int32)
        col1 = jnp.full((L,), 1, jnp.int32)
        col2 = jnp.full((L,), 2, jnp.int32)

        def chunk_body(j, _):
            base = base0 + j * chunk
            pltpu.sync_copy(src_hbm.at[pl.ds(base, chunk)], sidx)
            pltpu.sync_copy(dst_hbm.at[pl.ds(base, chunk)], didx)
            cps = pltpu.async_copy(coords_hbm.at[sidx], srow, sem_s)
            cpd = pltpu.async_copy(coords_hbm.at[didx], drow, sem_d)
            cps.wait()
            cpd.wait()

            def grp(g, _):
                rows = lax.iota(jnp.int32, L) + g * L
                sx = plsc.load_gather(srow, [rows, col0])
                sy = plsc.load_gather(srow, [rows, col1])
                sz = plsc.load_gather(srow, [rows, col2])
                tx = plsc.load_gather(drow, [rows, col0])
                ty = plsc.load_gather(drow, [rows, col1])
                tz = plsc.load_gather(drow, [rows, col2])
                vx = tx - sx
                vy = ty - sy
                vz = tz - sz
                d2 = vx * vx + vy * vy + vz * vz
                d2 = jnp.maximum(d2, 1e-12)
                i = plsc.bitcast(d2, jnp.int32)
                i = jnp.int32(0x5F3759DF) - (i >> 1)
                y = plsc.bitcast(i, jnp.float32)
                for _n in range(3):
                    y = y * (1.5 - 0.5 * d2 * y * y)
                r = d2 * y
                mask_b = d2 < cut2
                s = jnp.minimum(d2 * inv_cut2, 1.0)
                q = jnp.full((L,), _SW_C[6], jnp.float32)
                for c in (_SW_C[5], _SW_C[4], _SW_C[3], _SW_C[2],
                          _SW_C[1], _SW_C[0]):
                    q = q * s + c
                sw = jnp.where(mask_b, q, 0.0)
                mi = jnp.where(mask_b, 1, 0).astype(jnp.int32)

                plsc.store_scatter(vec_v, [rows, col0], vx)
                plsc.store_scatter(vec_v, [rows, col1], vy)
                plsc.store_scatter(vec_v, [rows, col2], vz)
                off = g * L
                dist_v[pl.ds(off, L)] = r
                sw_v[pl.ds(off, L)] = sw
                mask_v[pl.ds(off, L)] = mi
                return 0

            lax.fori_loop(0, groups, grp, 0)

            pltpu.sync_copy(vec_v, vec_hbm.at[pl.ds(base, chunk)])
            pltpu.sync_copy(dist_v, dist_hbm.at[pl.ds(base, chunk)])
            pltpu.sync_copy(sw_v, sw_hbm.at[pl.ds(base, chunk)])
            pltpu.sync_copy(mask_v, mask_hbm.at[pl.ds(base, chunk)])
            return 0

        lax.fori_loop(0, n_chunks, chunk_body, 0)

    return k


def kernel(coordinates, edge_src, edge_dst):
    n_nodes = coordinates.shape[0]
    n_edges = edge_src.shape[0]
    coords4 = jnp.concatenate(
        [coordinates, jnp.zeros((n_nodes, 1), jnp.float32)], axis=1)
    k = _make_kernel(n_nodes, n_edges, chunk=2000)
    vec, dist, sw, mask = k(coords4, edge_src, edge_dst)
    return vec, dist, sw, mask.astype(jnp.bool_)


# speedup vs baseline: 8.7643x; 8.7643x over previous
import functools

import jax
import jax.numpy as jnp
from jax import lax
from jax.experimental import pallas as pl
from jax.experimental.pallas import tpu as pltpu
from jax.experimental.pallas import tpu_sc as plsc

CUTOFF = 5.0
NC, NS, L = 2, 16, 16
NW = NC * NS

_COS_C = (
    0.9999999738948335,
    -4.934800732956998,
    4.058692224683156,
    -1.3351515271358487,
    0.2350219310960258,
    -0.02535563092319045,
    0.0015937868699634932,
)
_SW_C = tuple(0.5 * c for c in _COS_C)
_SW_C = (_SW_C[0] + 0.5,) + _SW_C[1:]


def _make_kernel(n_nodes: int, n_edges: int, chunk: int):
    assert n_edges % (NW * chunk) == 0
    e_per_w = n_edges // NW
    n_chunks = e_per_w // chunk
    groups = chunk // L
    mesh = plsc.VectorSubcoreMesh(core_axis_name="c", subcore_axis_name="s")
    inv_cut2 = 1.0 / (CUTOFF * CUTOFF)
    cut2 = CUTOFF * CUTOFF

    @functools.partial(
        pl.kernel,
        mesh=mesh,
        out_type=[
            jax.ShapeDtypeStruct((n_edges * 3,), jnp.float32),
            jax.ShapeDtypeStruct((n_edges,), jnp.float32),
            jax.ShapeDtypeStruct((n_edges,), jnp.float32),
            jax.ShapeDtypeStruct((n_edges,), jnp.int32),
        ],
        scratch_types=[
            pltpu.VMEM((chunk,), jnp.int32),
            pltpu.VMEM((chunk,), jnp.int32),
            pltpu.VMEM((chunk,), jnp.float32),
            pltpu.VMEM((chunk,), jnp.float32),
            pltpu.VMEM((chunk,), jnp.float32),
            pltpu.VMEM((chunk,), jnp.float32),
            pltpu.VMEM((chunk,), jnp.float32),
            pltpu.VMEM((chunk,), jnp.float32),
            pltpu.VMEM((chunk * 3,), jnp.float32),
            pltpu.VMEM((chunk,), jnp.float32),
            pltpu.VMEM((chunk,), jnp.float32),
            pltpu.VMEM((chunk,), jnp.int32),
            pltpu.SemaphoreType.DMA,
        ],
        compiler_params=pltpu.CompilerParams(needs_layout_passes=False),
    )
    def k(xs_hbm, ys_hbm, zs_hbm, src_hbm, dst_hbm,
          vec_hbm, dist_hbm, sw_hbm, mask_hbm,
          sidx, didx, sx_v, sy_v, sz_v, dx_v, dy_v, dz_v,
          vec_f, dist_v, sw_v, mask_v, sem):
        wid = lax.axis_index("s") * NC + lax.axis_index("c")
        base0 = wid * e_per_w
        vec_hbm_f = vec_hbm

        def chunk_body(j, _):
            base = base0 + j * chunk
            pltpu.sync_copy(src_hbm.at[pl.ds(base, chunk)], sidx)
            pltpu.sync_copy(dst_hbm.at[pl.ds(base, chunk)], didx)
            cps = [
                pltpu.async_copy(xs_hbm.at[sidx], sx_v, sem),
                pltpu.async_copy(ys_hbm.at[sidx], sy_v, sem),
                pltpu.async_copy(zs_hbm.at[sidx], sz_v, sem),
                pltpu.async_copy(xs_hbm.at[didx], dx_v, sem),
                pltpu.async_copy(ys_hbm.at[didx], dy_v, sem),
                pltpu.async_copy(zs_hbm.at[didx], dz_v, sem),
            ]
            for cp in cps:
                cp.wait()

            def grp(g, _):
                off = g * L
                sl = pl.ds(off, L)
                vx = dx_v[sl] - sx_v[sl]
                vy = dy_v[sl] - sy_v[sl]
                vz = dz_v[sl] - sz_v[sl]
                d2 = vx * vx + vy * vy + vz * vz
                d2 = jnp.maximum(d2, 1e-12)
                i = plsc.bitcast(d2, jnp.int32)
                i = jnp.int32(0x5F3759DF) - (i >> 1)
                y = plsc.bitcast(i, jnp.float32)
                for _n in range(3):
                    y = y * (1.5 - 0.5 * d2 * y * y)
                r = d2 * y
                mask_b = d2 < cut2
                s = jnp.minimum(d2 * inv_cut2, 1.0)
                q = jnp.full((L,), _SW_C[6], jnp.float32)
                for c in (_SW_C[5], _SW_C[4], _SW_C[3], _SW_C[2],
                          _SW_C[1], _SW_C[0]):
                    q = q * s + c
                sw = jnp.where(mask_b, q, 0.0)
                mi = jnp.where(mask_b, 1, 0).astype(jnp.int32)

                r3 = (lax.iota(jnp.int32, L) + off) * 3
                plsc.store_scatter(vec_f, [r3], vx)
                plsc.store_scatter(vec_f, [r3 + 1], vy)
                plsc.store_scatter(vec_f, [r3 + 2], vz)
                dist_v[sl] = r
                sw_v[sl] = sw
                mask_v[sl] = mi
                return 0

            lax.fori_loop(0, groups, grp, 0)

            pltpu.sync_copy(vec_f, vec_hbm_f.at[pl.ds(base * 3, chunk * 3)])
            pltpu.sync_copy(dist_v, dist_hbm.at[pl.ds(base, chunk)])
            pltpu.sync_copy(sw_v, sw_hbm.at[pl.ds(base, chunk)])
            pltpu.sync_copy(mask_v, mask_hbm.at[pl.ds(base, chunk)])
            return 0

        lax.fori_loop(0, n_chunks, chunk_body, 0)

    return k


def kernel(coordinates, edge_src, edge_dst):
    n_nodes = coordinates.shape[0]
    n_edges = edge_src.shape[0]
    xs = coordinates[:, 0]
    ys = coordinates[:, 1]
    zs = coordinates[:, 2]
    k = _make_kernel(n_nodes, n_edges, chunk=2000)
    vec, dist, sw, mask = k(xs, ys, zs, edge_src, edge_dst)
    return vec.reshape(n_edges, 3), dist, sw, mask.astype(jnp.bool_)


# speedup vs baseline: 26.9432x; 3.0742x over previous
import functools

import jax
import jax.numpy as jnp
from jax import lax
from jax.experimental import pallas as pl
from jax.experimental.pallas import tpu as pltpu
from jax.experimental.pallas import tpu_sc as plsc

CUTOFF = 5.0
NC, NS, L = 2, 16, 16
NW = NC * NS

_COS_C = (
    0.9999999738948335,
    -4.934800732956998,
    4.058692224683156,
    -1.3351515271358487,
    0.2350219310960258,
    -0.02535563092319045,
    0.0015937868699634932,
)
_SW_C = tuple(0.5 * c for c in _COS_C)
_SW_C = (_SW_C[0] + 0.5,) + _SW_C[1:]


def _make_kernel(n_nodes: int, n_edges: int, chunk: int):
    assert n_edges % (NW * chunk) == 0
    e_per_w = n_edges // NW
    n_chunks = e_per_w // chunk
    groups = chunk // L
    mesh = plsc.VectorSubcoreMesh(core_axis_name="c", subcore_axis_name="s")
    inv_cut2 = 1.0 / (CUTOFF * CUTOFF)
    cut2 = CUTOFF * CUTOFF

    @functools.partial(
        pl.kernel,
        mesh=mesh,
        out_type=[
            jax.ShapeDtypeStruct((n_edges,), jnp.float32),
            jax.ShapeDtypeStruct((n_edges,), jnp.float32),
            jax.ShapeDtypeStruct((n_edges,), jnp.float32),
            jax.ShapeDtypeStruct((n_edges,), jnp.float32),
            jax.ShapeDtypeStruct((n_edges,), jnp.float32),
            jax.ShapeDtypeStruct((n_edges,), jnp.int32),
        ],
        scratch_types=[
            pltpu.VMEM((chunk,), jnp.int32),
            pltpu.VMEM((chunk,), jnp.int32),
            pltpu.VMEM((chunk,), jnp.float32),
            pltpu.VMEM((chunk,), jnp.float32),
            pltpu.VMEM((chunk,), jnp.float32),
            pltpu.VMEM((chunk,), jnp.float32),
            pltpu.VMEM((chunk,), jnp.float32),
            pltpu.VMEM((chunk,), jnp.float32),
            pltpu.VMEM((chunk,), jnp.float32),
            pltpu.VMEM((chunk,), jnp.float32),
            pltpu.VMEM((chunk,), jnp.float32),
            pltpu.VMEM((chunk,), jnp.float32),
            pltpu.VMEM((chunk,), jnp.float32),
            pltpu.VMEM((chunk,), jnp.int32),
            pltpu.SemaphoreType.DMA,
        ],
        compiler_params=pltpu.CompilerParams(needs_layout_passes=False),
    )
    def k(xs_hbm, ys_hbm, zs_hbm, src_hbm, dst_hbm,
          vx_hbm, vy_hbm, vz_hbm, dist_hbm, sw_hbm, mask_hbm,
          sidx, didx, sx_v, sy_v, sz_v, dx_v, dy_v, dz_v,
          vx_v, vy_v, vz_v, dist_v, sw_v, mask_v, sem):
        wid = lax.axis_index("s") * NC + lax.axis_index("c")
        base0 = wid * e_per_w

        def chunk_body(j, _):
            base = base0 + j * chunk
            pltpu.sync_copy(src_hbm.at[pl.ds(base, chunk)], sidx)
            pltpu.sync_copy(dst_hbm.at[pl.ds(base, chunk)], didx)
            cps = [
                pltpu.async_copy(xs_hbm.at[sidx], sx_v, sem),
                pltpu.async_copy(ys_hbm.at[sidx], sy_v, sem),
                pltpu.async_copy(zs_hbm.at[sidx], sz_v, sem),
                pltpu.async_copy(xs_hbm.at[didx], dx_v, sem),
                pltpu.async_copy(ys_hbm.at[didx], dy_v, sem),
                pltpu.async_copy(zs_hbm.at[didx], dz_v, sem),
            ]
            for cp in cps:
                cp.wait()

            def grp(g, _):
                sl = pl.ds(g * L, L)
                vx = dx_v[sl] - sx_v[sl]
                vy = dy_v[sl] - sy_v[sl]
                vz = dz_v[sl] - sz_v[sl]
                d2 = vx * vx + vy * vy + vz * vz
                d2 = jnp.maximum(d2, 1e-12)
                i = plsc.bitcast(d2, jnp.int32)
                i = jnp.int32(0x5F3759DF) - (i >> 1)
                y = plsc.bitcast(i, jnp.float32)
                for _n in range(3):
                    y = y * (1.5 - 0.5 * d2 * y * y)
                r = d2 * y
                mask_b = d2 < cut2
                s = jnp.minimum(d2 * inv_cut2, 1.0)
                q = jnp.full((L,), _SW_C[6], jnp.float32)
                for c in (_SW_C[5], _SW_C[4], _SW_C[3], _SW_C[2],
                          _SW_C[1], _SW_C[0]):
                    q = q * s + c
                sw = jnp.where(mask_b, q, 0.0)
                mi = jnp.where(mask_b, 1, 0).astype(jnp.int32)

                vx_v[sl] = vx
                vy_v[sl] = vy
                vz_v[sl] = vz
                dist_v[sl] = r
                sw_v[sl] = sw
                mask_v[sl] = mi
                return 0

            lax.fori_loop(0, groups, grp, 0)

            pltpu.sync_copy(vx_v, vx_hbm.at[pl.ds(base, chunk)])
            pltpu.sync_copy(vy_v, vy_hbm.at[pl.ds(base, chunk)])
            pltpu.sync_copy(vz_v, vz_hbm.at[pl.ds(base, chunk)])
            pltpu.sync_copy(dist_v, dist_hbm.at[pl.ds(base, chunk)])
            pltpu.sync_copy(sw_v, sw_hbm.at[pl.ds(base, chunk)])
            pltpu.sync_copy(mask_v, mask_hbm.at[pl.ds(base, chunk)])
            return 0

        lax.fori_loop(0, n_chunks, chunk_body, 0)

    return k


def kernel(coordinates, edge_src, edge_dst):
    n_nodes = coordinates.shape[0]
    n_edges = edge_src.shape[0]
    xs = coordinates[:, 0]
    ys = coordinates[:, 1]
    zs = coordinates[:, 2]
    k = _make_kernel(n_nodes, n_edges, chunk=2000)
    vx, vy, vz, dist, sw, mask = k(xs, ys, zs, edge_src, edge_dst)
    vec = jnp.stack([vx, vy, vz], axis=-1)
    return vec, dist, sw, mask.astype(jnp.bool_)


# speedup vs baseline: 29.8211x; 1.1068x over previous
import functools

import jax
import jax.numpy as jnp
from jax import lax
from jax.experimental import pallas as pl
from jax.experimental.pallas import tpu as pltpu
from jax.experimental.pallas import tpu_sc as plsc

CUTOFF = 5.0
_CUT2_EDGE = 24.999998092651367
NC, NS, L = 2, 16, 16
NW = NC * NS

_COS_C = (
    0.9999999738948335,
    -4.934800732956998,
    4.058692224683156,
    -1.3351515271358487,
    0.2350219310960258,
    -0.02535563092319045,
    0.0015937868699634932,
)
_SW_C = tuple(0.5 * c for c in _COS_C)
_SW_C = (_SW_C[0] + 0.5,) + _SW_C[1:]


def _make_kernel(n_nodes: int, n_edges: int, chunk: int):
    assert n_edges % (NW * chunk) == 0
    e_per_w = n_edges // NW
    n_chunks = e_per_w // chunk
    assert n_chunks % 2 == 0
    groups = chunk // L
    mesh = plsc.VectorSubcoreMesh(core_axis_name="c", subcore_axis_name="s")
    inv_cut2 = 1.0 / (CUTOFF * CUTOFF)

    gather_set = [
        pltpu.VMEM((chunk,), jnp.int32),
        pltpu.VMEM((chunk,), jnp.int32),
        pltpu.VMEM((chunk,), jnp.float32),
        pltpu.VMEM((chunk,), jnp.float32),
        pltpu.VMEM((chunk,), jnp.float32),
        pltpu.VMEM((chunk,), jnp.float32),
        pltpu.VMEM((chunk,), jnp.float32),
        pltpu.VMEM((chunk,), jnp.float32),
    ]
    out_set = [
        pltpu.VMEM((chunk,), jnp.float32),
        pltpu.VMEM((chunk,), jnp.float32),
        pltpu.VMEM((chunk,), jnp.float32),
        pltpu.VMEM((chunk,), jnp.float32),
        pltpu.VMEM((chunk,), jnp.float32),
        pltpu.VMEM((chunk,), jnp.int32),
    ]

    @functools.partial(
        pl.kernel,
        mesh=mesh,
        out_type=[
            jax.ShapeDtypeStruct((n_edges,), jnp.float32),
            jax.ShapeDtypeStruct((n_edges,), jnp.float32),
            jax.ShapeDtypeStruct((n_edges,), jnp.float32),
            jax.ShapeDtypeStruct((n_edges,), jnp.float32),
            jax.ShapeDtypeStruct((n_edges,), jnp.float32),
            jax.ShapeDtypeStruct((n_edges,), jnp.int32),
        ],
        scratch_types=(gather_set + gather_set + out_set + out_set + [
            pltpu.SemaphoreType.DMA,
            pltpu.SemaphoreType.DMA,
            pltpu.SemaphoreType.DMA,
            pltpu.SemaphoreType.DMA,
        ]),
        compiler_params=pltpu.CompilerParams(needs_layout_passes=False),
    )
    def k(xs_hbm, ys_hbm, zs_hbm, src_hbm, dst_hbm,
          vx_hbm, vy_hbm, vz_hbm, dist_hbm, sw_hbm, mask_hbm,
          *bufs):
        ga = bufs[0:8]
        gb = bufs[8:16]
        oa = bufs[16:22]
        ob = bufs[22:28]
        sem_ga, sem_gb, sem_oa, sem_ob = bufs[28:32]
        out_hbms = (vx_hbm, vy_hbm, vz_hbm, dist_hbm, sw_hbm, mask_hbm)

        wid = lax.axis_index("s") * NC + lax.axis_index("c")
        base0 = wid * e_per_w

        def start_gathers(j, bufset, sem):
            base = base0 + j * chunk
            sidx, didx, sx, sy, sz, dx, dy, dz = bufset
            pltpu.sync_copy(src_hbm.at[pl.ds(base, chunk)], sidx)
            pltpu.sync_copy(dst_hbm.at[pl.ds(base, chunk)], didx)
            pltpu.async_copy(xs_hbm.at[sidx], sx, sem)
            pltpu.async_copy(ys_hbm.at[sidx], sy, sem)
            pltpu.async_copy(zs_hbm.at[sidx], sz, sem)
            pltpu.async_copy(xs_hbm.at[didx], dx, sem)
            pltpu.async_copy(ys_hbm.at[didx], dy, sem)
            pltpu.async_copy(zs_hbm.at[didx], dz, sem)

        def wait_gathers(bufset, sem):
            sidx, didx, sx, sy, sz, dx, dy, dz = bufset
            for dst in (sx, sy, sz, dx, dy, dz):
                pltpu.make_async_copy(xs_hbm.at[sidx], dst, sem).wait()

        def wait_outs(j, outset, sem):
            base = base0 + j * chunk
            for src, hbm in zip(outset, out_hbms):
                pltpu.make_async_copy(src, hbm.at[pl.ds(base, chunk)],
                                      sem).wait()

        def start_outs(j, outset, sem):
            base = base0 + j * chunk
            for src, hbm in zip(outset, out_hbms):
                pltpu.async_copy(src, hbm.at[pl.ds(base, chunk)], sem)

        def compute(bufset, outset):
            _, _, sx_v, sy_v, sz_v, dx_v, dy_v, dz_v = bufset
            vx_v, vy_v, vz_v, dist_v, sw_v, mask_v = outset

            def grp(g, _):
                sl = pl.ds(g * L, L)
                vx = dx_v[sl] - sx_v[sl]
                vy = dy_v[sl] - sy_v[sl]
                vz = dz_v[sl] - sz_v[sl]
                d2 = vx * vx + vy * vy + vz * vz
                d2 = jnp.maximum(d2, 1e-12)
                i = plsc.bitcast(d2, jnp.int32)
                i = jnp.int32(0x5F3759DF) - (i >> 1)
                y = plsc.bitcast(i, jnp.float32)
                for _n in range(3):
                    y = y * (1.5 - 0.5 * d2 * y * y)
                r = d2 * y
                mask_b = d2 < _CUT2_EDGE
                s = jnp.minimum(d2 * inv_cut2, 1.0)
                q = jnp.full((L,), _SW_C[6], jnp.float32)
                for c in (_SW_C[5], _SW_C[4], _SW_C[3], _SW_C[2],
                          _SW_C[1], _SW_C[0]):
                    q = q * s + c
                sw = jnp.where(mask_b, q, 0.0)
                mi = jnp.where(mask_b, 1, 0).astype(jnp.int32)

                vx_v[sl] = vx
                vy_v[sl] = vy
                vz_v[sl] = vz
                dist_v[sl] = r
                sw_v[sl] = sw
                mask_v[sl] = mi
                return 0

            lax.fori_loop(0, groups, grp, 0)

        start_gathers(0, ga, sem_ga)

        def pair_body(p, _):
            j0 = 2 * p
            j1 = j0 + 1
            start_gathers(j1, gb, sem_gb)
            wait_gathers(ga, sem_ga)

            @pl.when(p > 0)
            def _():
                wait_outs(j0 - 2, oa, sem_oa)
            compute(ga, oa)
            start_outs(j0, oa, sem_oa)

            @pl.when(j1 + 1 < n_chunks)
            def _():
                start_gathers(j1 + 1, ga, sem_ga)
            wait_gathers(gb, sem_gb)

            @pl.when(p > 0)
            def _():
                wait_outs(j1 - 2, ob, sem_ob)
            compute(gb, ob)
            start_outs(j1, ob, sem_ob)
            return 0

        lax.fori_loop(0, n_chunks // 2, pair_body, 0)
        wait_outs(n_chunks - 2, oa, sem_oa)
        wait_outs(n_chunks - 1, ob, sem_ob)

    return k


def kernel(coordinates, edge_src, edge_dst):
    n_nodes = coordinates.shape[0]
    n_edges = edge_src.shape[0]
    xs = coordinates[:, 0]
    ys = coordinates[:, 1]
    zs = coordinates[:, 2]
    k = _make_kernel(n_nodes, n_edges, chunk=2000)
    vx, vy, vz, dist, sw, mask = k(xs, ys, zs, edge_src, edge_dst)
    vec = jnp.stack([vx, vy, vz], axis=-1)
    return vec, dist, sw, mask.astype(jnp.bool_)


# speedup vs baseline: 106.3128x; 3.5650x over previous
import functools

import jax
import jax.numpy as jnp
from jax import lax
from jax.experimental import pallas as pl
from jax.experimental.pallas import tpu as pltpu
from jax.experimental.pallas import tpu_sc as plsc

CUTOFF = 5.0
_CUT2_EDGE = 24.999998092651367
NC, NS, L = 2, 16, 16
NW = NC * NS

_COS_C = (
    0.9999999738948335,
    -4.934800732956998,
    4.058692224683156,
    -1.3351515271358487,
    0.2350219310960258,
    -0.02535563092319045,
    0.0015937868699634932,
)
_SW_C = tuple(0.5 * c for c in _COS_C)
_SW_C = (_SW_C[0] + 0.5,) + _SW_C[1:]


def _make_kernel(n_nodes: int, n_edges: int, chunk: int):
    assert n_edges % (NW * chunk) == 0
    e_per_w = n_edges // NW
    n_chunks = e_per_w // chunk
    assert n_chunks % 2 == 0
    groups = chunk // L
    mesh = plsc.VectorSubcoreMesh(core_axis_name="c", subcore_axis_name="s")
    inv_cut2 = 1.0 / (CUTOFF * CUTOFF)

    gather_set = [
        pltpu.VMEM((chunk,), jnp.int32),
        pltpu.VMEM((chunk,), jnp.int32),
        pltpu.VMEM((chunk,), jnp.float32),
        pltpu.VMEM((chunk,), jnp.float32),
        pltpu.VMEM((chunk,), jnp.float32),
        pltpu.VMEM((chunk,), jnp.float32),
        pltpu.VMEM((chunk,), jnp.float32),
        pltpu.VMEM((chunk,), jnp.float32),
    ]
    out_set = [
        pltpu.VMEM((chunk,), jnp.float32),
        pltpu.VMEM((chunk,), jnp.float32),
        pltpu.VMEM((chunk,), jnp.float32),
        pltpu.VMEM((chunk,), jnp.float32),
        pltpu.VMEM((chunk,), jnp.float32),
        pltpu.VMEM((chunk,), jnp.int32),
    ]

    @functools.partial(
        pl.kernel,
        mesh=mesh,
        out_type=[
            jax.ShapeDtypeStruct((n_edges,), jnp.float32),
            jax.ShapeDtypeStruct((n_edges,), jnp.float32),
            jax.ShapeDtypeStruct((n_edges,), jnp.float32),
            jax.ShapeDtypeStruct((n_edges,), jnp.float32),
            jax.ShapeDtypeStruct((n_edges,), jnp.float32),
            jax.ShapeDtypeStruct((n_edges,), jnp.int32),
        ],
        scratch_types=(gather_set + gather_set + out_set + out_set + [
            pltpu.VMEM_SHARED((n_nodes,), jnp.float32),
            pltpu.VMEM_SHARED((n_nodes,), jnp.float32),
            pltpu.VMEM_SHARED((n_nodes,), jnp.float32),
            pltpu.SemaphoreType.DMA,
            pltpu.SemaphoreType.DMA,
            pltpu.SemaphoreType.DMA,
            pltpu.SemaphoreType.DMA,
        ]),
        compiler_params=pltpu.CompilerParams(needs_layout_passes=False),
    )
    def k(xs_hbm, ys_hbm, zs_hbm, src_hbm, dst_hbm,
          vx_hbm, vy_hbm, vz_hbm, dist_hbm, sw_hbm, mask_hbm,
          *bufs):
        ga = bufs[0:8]
        gb = bufs[8:16]
        oa = bufs[16:22]
        ob = bufs[22:28]
        xs_s, ys_s, zs_s = bufs[28:31]
        sem_ga, sem_gb, sem_oa, sem_ob = bufs[31:35]
        out_hbms = (vx_hbm, vy_hbm, vz_hbm, dist_hbm, sw_hbm, mask_hbm)

        wid = lax.axis_index("s") * NC + lax.axis_index("c")
        base0 = wid * e_per_w

        @pl.when(lax.axis_index("s") == 0)
        def _():
            pltpu.sync_copy(xs_hbm, xs_s)
            pltpu.sync_copy(ys_hbm, ys_s)
            pltpu.sync_copy(zs_hbm, zs_s)
        plsc.subcore_barrier()

        def start_gathers(j, bufset, sem):
            base = base0 + j * chunk
            sidx, didx, sx, sy, sz, dx, dy, dz = bufset
            pltpu.sync_copy(src_hbm.at[pl.ds(base, chunk)], sidx)
            pltpu.sync_copy(dst_hbm.at[pl.ds(base, chunk)], didx)
            pltpu.async_copy(xs_s.at[sidx], sx, sem)
            pltpu.async_copy(ys_s.at[sidx], sy, sem)
            pltpu.async_copy(zs_s.at[sidx], sz, sem)
            pltpu.async_copy(xs_s.at[didx], dx, sem)
            pltpu.async_copy(ys_s.at[didx], dy, sem)
            pltpu.async_copy(zs_s.at[didx], dz, sem)

        def wait_gathers(bufset, sem):
            sidx, didx, sx, sy, sz, dx, dy, dz = bufset
            for dst in (sx, sy, sz, dx, dy, dz):
                pltpu.make_async_copy(xs_s.at[sidx], dst, sem).wait()

        def wait_outs(j, outset, sem):
            base = base0 + j * chunk
            for src, hbm in zip(outset, out_hbms):
                pltpu.make_async_copy(src, hbm.at[pl.ds(base, chunk)],
                                      sem).wait()

        def start_outs(j, outset, sem):
            base = base0 + j * chunk
            for src, hbm in zip(outset, out_hbms):
                pltpu.async_copy(src, hbm.at[pl.ds(base, chunk)], sem)

        def compute(bufset, outset):
            _, _, sx_v, sy_v, sz_v, dx_v, dy_v, dz_v = bufset
            vx_v, vy_v, vz_v, dist_v, sw_v, mask_v = outset

            def grp(g, _):
                sl = pl.ds(g * L, L)
                vx = dx_v[sl] - sx_v[sl]
                vy = dy_v[sl] - sy_v[sl]
                vz = dz_v[sl] - sz_v[sl]
                d2 = vx * vx + vy * vy + vz * vz
                d2 = jnp.maximum(d2, 1e-12)
                i = plsc.bitcast(d2, jnp.int32)
                i = jnp.int32(0x5F3759DF) - (i >> 1)
                y = plsc.bitcast(i, jnp.float32)
                for _n in range(3):
                    y = y * (1.5 - 0.5 * d2 * y * y)
                r = d2 * y
                mask_b = d2 < _CUT2_EDGE
                s = jnp.minimum(d2 * inv_cut2, 1.0)
                q = jnp.full((L,), _SW_C[6], jnp.float32)
                for c in (_SW_C[5], _SW_C[4], _SW_C[3], _SW_C[2],
                          _SW_C[1], _SW_C[0]):
                    q = q * s + c
                sw = jnp.where(mask_b, q, 0.0)
                mi = jnp.where(mask_b, 1, 0).astype(jnp.int32)

                vx_v[sl] = vx
                vy_v[sl] = vy
                vz_v[sl] = vz
                dist_v[sl] = r
                sw_v[sl] = sw
                mask_v[sl] = mi
                return 0

            lax.fori_loop(0, groups, grp, 0)

        start_gathers(0, ga, sem_ga)

        def pair_body(p, _):
            j0 = 2 * p
            j1 = j0 + 1
            start_gathers(j1, gb, sem_gb)
            wait_gathers(ga, sem_ga)

            @pl.when(p > 0)
            def _():
                wait_outs(j0 - 2, oa, sem_oa)
            compute(ga, oa)
            start_outs(j0, oa, sem_oa)

            @pl.when(j1 + 1 < n_chunks)
            def _():
                start_gathers(j1 + 1, ga, sem_ga)
            wait_gathers(gb, sem_gb)

            @pl.when(p > 0)
            def _():
                wait_outs(j1 - 2, ob, sem_ob)
            compute(gb, ob)
            start_outs(j1, ob, sem_ob)
            return 0

        lax.fori_loop(0, n_chunks // 2, pair_body, 0)
        wait_outs(n_chunks - 2, oa, sem_oa)
        wait_outs(n_chunks - 1, ob, sem_ob)

    return k


def kernel(coordinates, edge_src, edge_dst):
    n_nodes = coordinates.shape[0]
    n_edges = edge_src.shape[0]
    xs = coordinates[:, 0]
    ys = coordinates[:, 1]
    zs = coordinates[:, 2]
    k = _make_kernel(n_nodes, n_edges, chunk=2000)
    vx, vy, vz, dist, sw, mask = k(xs, ys, zs, edge_src, edge_dst)
    vec = jnp.stack([vx, vy, vz], axis=-1)
    return vec, dist, sw, mask.astype(jnp.bool_)


# speedup vs baseline: 106.5961x; 1.0027x over previous
import functools

import jax
import jax.numpy as jnp
from jax import lax
from jax.experimental import pallas as pl
from jax.experimental.pallas import tpu as pltpu
from jax.experimental.pallas import tpu_sc as plsc

CUTOFF = 5.0
_CUT2_EDGE = 24.999998092651367
NC, NS, L = 2, 16, 16
NW = NC * NS

_COS_C = (
    0.9999999738948335,
    -4.934800732956998,
    4.058692224683156,
    -1.3351515271358487,
    0.2350219310960258,
    -0.02535563092319045,
    0.0015937868699634932,
)
_SW_C = tuple(0.5 * c for c in _COS_C)
_SW_C = (_SW_C[0] + 0.5,) + _SW_C[1:]


def _make_kernel(n_nodes: int, n_edges: int, chunk: int):
    assert n_edges % (NW * chunk) == 0
    e_per_w = n_edges // NW
    n_chunks = e_per_w // chunk
    assert n_chunks % 2 == 0
    groups = chunk // L
    mesh = plsc.VectorSubcoreMesh(core_axis_name="c", subcore_axis_name="s")
    inv_cut2 = 1.0 / (CUTOFF * CUTOFF)

    gather_set = [
        pltpu.VMEM((chunk,), jnp.int32),
        pltpu.VMEM((chunk,), jnp.int32),
        pltpu.VMEM((chunk,), jnp.float32),
        pltpu.VMEM((chunk,), jnp.float32),
        pltpu.VMEM((chunk,), jnp.float32),
        pltpu.VMEM((chunk,), jnp.float32),
        pltpu.VMEM((chunk,), jnp.float32),
        pltpu.VMEM((chunk,), jnp.float32),
    ]
    out_set = [
        pltpu.VMEM((chunk,), jnp.float32),
        pltpu.VMEM((chunk,), jnp.float32),
        pltpu.VMEM((chunk,), jnp.float32),
        pltpu.VMEM((chunk,), jnp.float32),
        pltpu.VMEM((chunk,), jnp.float32),
        pltpu.VMEM((chunk,), jnp.int32),
    ]

    @functools.partial(
        pl.kernel,
        mesh=mesh,
        out_type=[
            jax.ShapeDtypeStruct((n_edges,), jnp.float32),
            jax.ShapeDtypeStruct((n_edges,), jnp.float32),
            jax.ShapeDtypeStruct((n_edges,), jnp.float32),
            jax.ShapeDtypeStruct((n_edges,), jnp.float32),
            jax.ShapeDtypeStruct((n_edges,), jnp.float32),
            jax.ShapeDtypeStruct((n_edges,), jnp.int32),
        ],
        scratch_types=(gather_set + gather_set + out_set + out_set + [
            pltpu.VMEM_SHARED((n_nodes,), jnp.float32),
            pltpu.VMEM_SHARED((n_nodes,), jnp.float32),
            pltpu.VMEM_SHARED((n_nodes,), jnp.float32),
            pltpu.SemaphoreType.DMA,
            pltpu.SemaphoreType.DMA,
            pltpu.SemaphoreType.DMA,
            pltpu.SemaphoreType.DMA,
            pltpu.SemaphoreType.DMA,
            pltpu.SemaphoreType.DMA,
        ]),
        compiler_params=pltpu.CompilerParams(needs_layout_passes=False),
    )
    def k(xs_hbm, ys_hbm, zs_hbm, src_hbm, dst_hbm,
          vx_hbm, vy_hbm, vz_hbm, dist_hbm, sw_hbm, mask_hbm,
          *bufs):
        ga = bufs[0:8]
        gb = bufs[8:16]
        oa = bufs[16:22]
        ob = bufs[22:28]
        xs_s, ys_s, zs_s = bufs[28:31]
        sem_ga, sem_gb, sem_oa, sem_ob, sem_ia, sem_ib = bufs[31:37]
        out_hbms = (vx_hbm, vy_hbm, vz_hbm, dist_hbm, sw_hbm, mask_hbm)

        wid = lax.axis_index("s") * NC + lax.axis_index("c")
        base0 = wid * e_per_w

        @pl.when(lax.axis_index("s") == 0)
        def _():
            pltpu.sync_copy(xs_hbm, xs_s)
            pltpu.sync_copy(ys_hbm, ys_s)
            pltpu.sync_copy(zs_hbm, zs_s)
        plsc.subcore_barrier()

        def start_idx(j, bufset, sem):
            base = base0 + j * chunk
            sidx, didx = bufset[0], bufset[1]
            pltpu.async_copy(src_hbm.at[pl.ds(base, chunk)], sidx, sem)
            pltpu.async_copy(dst_hbm.at[pl.ds(base, chunk)], didx, sem)

        def wait_idx(bufset, sem):
            sidx, didx = bufset[0], bufset[1]
            pltpu.make_async_copy(src_hbm.at[pl.ds(0, chunk)], sidx,
                                  sem).wait()
            pltpu.make_async_copy(src_hbm.at[pl.ds(0, chunk)], didx,
                                  sem).wait()

        def start_gathers(bufset, sem):
            sidx, didx, sx, sy, sz, dx, dy, dz = bufset
            pltpu.async_copy(xs_s.at[sidx], sx, sem)
            pltpu.async_copy(ys_s.at[sidx], sy, sem)
            pltpu.async_copy(zs_s.at[sidx], sz, sem)
            pltpu.async_copy(xs_s.at[didx], dx, sem)
            pltpu.async_copy(ys_s.at[didx], dy, sem)
            pltpu.async_copy(zs_s.at[didx], dz, sem)

        def wait_gathers(bufset, sem):
            sidx, didx, sx, sy, sz, dx, dy, dz = bufset
            for dst in (sx, sy, sz, dx, dy, dz):
                pltpu.make_async_copy(xs_s.at[sidx], dst, sem).wait()

        def wait_outs(j, outset, sem):
            base = base0 + j * chunk
            for src, hbm in zip(outset, out_hbms):
                pltpu.make_async_copy(src, hbm.at[pl.ds(base, chunk)],
                                      sem).wait()

        def start_outs(j, outset, sem):
            base = base0 + j * chunk
            for src, hbm in zip(outset, out_hbms):
                pltpu.async_copy(src, hbm.at[pl.ds(base, chunk)], sem)

        def compute(bufset, outset):
            _, _, sx_v, sy_v, sz_v, dx_v, dy_v, dz_v = bufset
            vx_v, vy_v, vz_v, dist_v, sw_v, mask_v = outset

            def grp(g, _):
                sl = pl.ds(g * L, L)
                vx = dx_v[sl] - sx_v[sl]
                vy = dy_v[sl] - sy_v[sl]
                vz = dz_v[sl] - sz_v[sl]
                d2 = vx * vx + vy * vy + vz * vz
                d2 = jnp.maximum(d2, 1e-12)
                i = plsc.bitcast(d2, jnp.int32)
                i = jnp.int32(0x5F3759DF) - (i >> 1)
                y = plsc.bitcast(i, jnp.float32)
                for _n in range(3):
                    y = y * (1.5 - 0.5 * d2 * y * y)
                r = d2 * y
                mask_b = d2 < _CUT2_EDGE
                s = jnp.minimum(d2 * inv_cut2, 1.0)
                q = jnp.full((L,), _SW_C[6], jnp.float32)
                for c in (_SW_C[5], _SW_C[4], _SW_C[3], _SW_C[2],
                          _SW_C[1], _SW_C[0]):
                    q = q * s + c
                sw = jnp.where(mask_b, q, 0.0)
                mi = jnp.where(mask_b, 1, 0).astype(jnp.int32)

                vx_v[sl] = vx
                vy_v[sl] = vy
                vz_v[sl] = vz
                dist_v[sl] = r
                sw_v[sl] = sw
                mask_v[sl] = mi
                return 0

            lax.fori_loop(0, groups, grp, 0)

        start_idx(0, ga, sem_ia)
        wait_idx(ga, sem_ia)
        start_gathers(ga, sem_ga)
        start_idx(1, gb, sem_ib)

        def pair_body(p, _):
            j0 = 2 * p
            j1 = j0 + 1
            wait_idx(gb, sem_ib)
            start_gathers(gb, sem_gb)
            wait_gathers(ga, sem_ga)

            @pl.when(j0 + 2 < n_chunks)
            def _():
                start_idx(j0 + 2, ga, sem_ia)

            @pl.when(p > 0)
            def _():
                wait_outs(j0 - 2, oa, sem_oa)
            compute(ga, oa)
            start_outs(j0, oa, sem_oa)

            @pl.when(j0 + 2 < n_chunks)
            def _():
                wait_idx(ga, sem_ia)
                start_gathers(ga, sem_ga)
            wait_gathers(gb, sem_gb)

            @pl.when(j1 + 2 < n_chunks)
            def _():
                start_idx(j1 + 2, gb, sem_ib)

            @pl.when(p > 0)
            def _():
                wait_outs(j1 - 2, ob, sem_ob)
            compute(gb, ob)
            start_outs(j1, ob, sem_ob)
            return 0

        lax.fori_loop(0, n_chunks // 2, pair_body, 0)
        wait_outs(n_chunks - 2, oa, sem_oa)
        wait_outs(n_chunks - 1, ob, sem_ob)

    return k


def kernel(coordinates, edge_src, edge_dst):
    n_nodes = coordinates.shape[0]
    n_edges = edge_src.shape[0]
    xs = coordinates[:, 0]
    ys = coordinates[:, 1]
    zs = coordinates[:, 2]
    k = _make_kernel(n_nodes, n_edges, chunk=2000)
    vx, vy, vz, dist, sw, mask = k(xs, ys, zs, edge_src, edge_dst)
    vec = jnp.stack([vx, vy, vz], axis=-1)
    return vec, dist, sw, mask.astype(jnp.bool_)
